# Initial kernel scaffold; baseline (speedup 1.0000x reference)
import jax
import jax.numpy as jnp
from jax.experimental import pallas as pl


def kernel(node_features, senders, receivers, W_fs, b_fs, W_gn, b_gn, W_gin, b_gin):
    raise NotImplementedError("write your pallas kernel here")


# speedup vs baseline: 1.0309x; 1.0309x over previous
import functools

import jax
import jax.numpy as jnp
from jax import lax
from jax.experimental import pallas as pl

N_NODES = 10000
BS = 1000


def _mm1_body(x_ref, wfs_ref, bfs_ref, wgn_ref, bgn_ref, t_ref, base_ref):
    x = x_ref[...]
    t_ref[...] = jnp.maximum(
        lax.dot_general(x, wfs_ref[...], (((1,), (1,)), ((), ()))) + bfs_ref[...], 0.0
    )
    base_ref[...] = lax.dot_general(x, wgn_ref[...], (((1,), (1,)), ((), ()))) + bgn_ref[...]


def _mm2_body(seg_ref, wgin_ref, bgin_ref, base_ref, out_ref):
    out_ref[...] = (
        lax.dot_general(seg_ref[...], wgin_ref[...], (((1,), (1,)), ((), ())))
        + bgin_ref[...]
        + base_ref[...]
    )


def kernel(node_features, senders, receivers, W_fs, b_fs, W_gn, b_gn, W_gin, b_gin):
    nb = N_NODES // BS
    transformed, base = pl.pallas_call(
        _mm1_body,
        grid=(nb,),
        in_specs=[
            pl.BlockSpec((BS, 128), lambda i: (i, 0)),
            pl.BlockSpec((128, 128), lambda i: (0, 0)),
            pl.BlockSpec((128,), lambda i: (0,)),
            pl.BlockSpec((128, 128), lambda i: (0, 0)),
            pl.BlockSpec((128,), lambda i: (0,)),
        ],
        out_specs=[
            pl.BlockSpec((BS, 128), lambda i: (i, 0)),
            pl.BlockSpec((BS, 128), lambda i: (i, 0)),
        ],
        out_shape=[
            jax.ShapeDtypeStruct((N_NODES, 128), jnp.float32),
            jax.ShapeDtypeStruct((N_NODES, 128), jnp.float32),
        ],
    )(node_features, W_fs, b_fs, W_gn, b_gn)

    seg = jax.ops.segment_max(jnp.take(transformed, senders, axis=0), receivers,
                              num_segments=N_NODES)
    seg = jnp.maximum(seg, 0.0)

    nodes = pl.pallas_call(
        _mm2_body,
        grid=(nb,),
        in_specs=[
            pl.BlockSpec((BS, 128), lambda i: (i, 0)),
            pl.BlockSpec((128, 128), lambda i: (0, 0)),
            pl.BlockSpec((128,), lambda i: (0,)),
            pl.BlockSpec((BS, 128), lambda i: (i, 0)),
        ],
        out_specs=pl.BlockSpec((BS, 128), lambda i: (i, 0)),
        out_shape=jax.ShapeDtypeStruct((N_NODES, 128), jnp.float32),
    )(seg, W_gin, b_gin, base)
    return nodes
